# Initial kernel scaffold; baseline (speedup 1.0000x reference)
import jax
import jax.numpy as jnp
from jax.experimental import pallas as pl


def kernel(x, edge_index, edge_attr, u, batch, params):
    raise NotImplementedError("write your pallas kernel here")


# speedup vs baseline: 3.7763x; 3.7763x over previous
import functools

import jax
import jax.numpy as jnp
from jax import lax
from jax.experimental import pallas as pl
from jax.experimental.pallas import tpu as pltpu
from jax.experimental.pallas import tpu_sc as plsc

F32 = jnp.float32
I32 = jnp.int32

NC = 2
NS = 16
NW = NC * NS

TN = 2000
TE = 8000
CH = 1000

_SC_PARAMS = pltpu.CompilerParams(use_tc_tiling_on_sc=False)
_SC_PARAMS_BIG = pltpu.CompilerParams(use_tc_tiling_on_sc=False,
                                      internal_scratch_in_bytes=0)


def _relu(v):
    return jnp.maximum(v, 0.0)


BF16 = jnp.bfloat16


def _dot(a, b):
    return jnp.dot(a, b, preferred_element_type=F32)


def _mlp2(v, w1, b1, w2, b2):
    return _dot(_relu(_dot(v, w1) + b1), w2) + b2


def _sc_gather(xtab, ubtab, row, col):
    e = row.shape[0]
    ew = e // NW
    nch = ew // CH
    mesh = plsc.VectorSubcoreMesh(core_axis_name="c", subcore_axis_name="s")
    out_t = jax.ShapeDtypeStruct((e, 32), BF16)

    @functools.partial(
        pl.kernel, mesh=mesh,
        compiler_params=_SC_PARAMS,
        out_type=(out_t, out_t, out_t),
        scratch_types=[
            pltpu.VMEM((CH,), I32), pltpu.VMEM((CH,), I32),
            pltpu.VMEM((CH, 32), BF16), pltpu.VMEM((CH, 32), BF16),
            pltpu.VMEM((CH, 32), BF16), pltpu.SemaphoreType.DMA,
        ],
    )
    def k(xt_hbm, ub_hbm, row_hbm, col_hbm, gxr_hbm, gub_hbm, gxc_hbm,
          idxr, idxc, bxr, bub, bxc, sem):
        wid = lax.axis_index("s") * NC + lax.axis_index("c")
        base = wid * ew

        @pl.loop(0, nch)
        def _(kk):
            off = base + kk * CH
            pltpu.sync_copy(row_hbm.at[pl.ds(off, CH)], idxr)
            pltpu.sync_copy(col_hbm.at[pl.ds(off, CH)], idxc)
            c1 = pltpu.async_copy(xt_hbm.at[idxr], bxr, sem)
            c2 = pltpu.async_copy(ub_hbm.at[idxr], bub, sem)
            c3 = pltpu.async_copy(xt_hbm.at[idxc], bxc, sem)
            c1.wait()
            c2.wait()
            c3.wait()
            pltpu.sync_copy(bxr, gxr_hbm.at[pl.ds(off, CH)])
            pltpu.sync_copy(bub, gub_hbm.at[pl.ds(off, CH)])
            pltpu.sync_copy(bxc, gxc_hbm.at[pl.ds(off, CH)])

    return k(xtab, ubtab, row, col)


def _sc_scatter_add(vals, row, n, zeros32):
    e = vals.shape[0]
    ew = e // NW
    chs = 200
    nch = ew // chs
    nps = n // NS
    mesh = plsc.VectorSubcoreMesh(core_axis_name="c", subcore_axis_name="s")

    @functools.partial(
        pl.kernel, mesh=mesh,
        compiler_params=_SC_PARAMS_BIG,
        out_type=jax.ShapeDtypeStruct((NC, n, 32), F32),
        scratch_types=[
            pltpu.VMEM((chs,), I32), pltpu.VMEM((chs, 32), F32),
            pltpu.VMEM_SHARED((n, 32), F32),
        ],
    )
    def k(vals_hbm, row_hbm, z_hbm, out_hbm, idx_v, val_v, shared):
        cid = lax.axis_index("c")
        sid = lax.axis_index("s")
        pltpu.sync_copy(z_hbm.at[pl.ds(sid * nps, nps)],
                        shared.at[pl.ds(sid * nps, nps)])
        plsc.subcore_barrier()
        base = cid * (ew * NS) + sid * ew

        @pl.loop(0, nch)
        def _(kk):
            off = base + kk * chs
            pltpu.sync_copy(row_hbm.at[pl.ds(off, chs)], idx_v)
            pltpu.sync_copy(vals_hbm.at[pl.ds(off, chs)], val_v)
            pltpu.sync_copy(val_v, shared.at[idx_v], add=True)

        plsc.subcore_barrier()
        pltpu.sync_copy(shared.at[pl.ds(sid * nps, nps)],
                        out_hbm.at[cid, pl.ds(sid * nps, nps)])

    return k(vals, row, zeros32)


def _sc_counts(row, n, zeros16, ones16):
    e = row.shape[0]
    ew = e // NW
    nch = ew // CH
    nps = n // NS
    mesh = plsc.VectorSubcoreMesh(core_axis_name="c", subcore_axis_name="s")

    @functools.partial(
        pl.kernel, mesh=mesh,
        compiler_params=_SC_PARAMS,
        out_type=jax.ShapeDtypeStruct((NC, n, 16), F32),
        scratch_types=[
            pltpu.VMEM((CH,), I32), pltpu.VMEM((CH, 16), F32),
            pltpu.VMEM_SHARED((n, 16), F32),
        ],
    )
    def k(row_hbm, z_hbm, ones_hbm, out_hbm, idx_v, ones_v, shared):
        cid = lax.axis_index("c")
        sid = lax.axis_index("s")
        pltpu.sync_copy(z_hbm.at[pl.ds(sid * nps, nps)],
                        shared.at[pl.ds(sid * nps, nps)])
        pltpu.sync_copy(ones_hbm, ones_v)
        plsc.subcore_barrier()
        base = cid * (ew * NS) + sid * ew

        @pl.loop(0, nch)
        def _(kk):
            off = base + kk * CH
            pltpu.sync_copy(row_hbm.at[pl.ds(off, CH)], idx_v)
            pltpu.sync_copy(ones_v, shared.at[idx_v], add=True)

        plsc.subcore_barrier()
        pltpu.sync_copy(shared.at[pl.ds(sid * nps, nps)],
                        out_hbm.at[cid, pl.ds(sid * nps, nps)])

    return k(row, zeros16, ones16)


def _tc_prep(x_in, u_src, batch_col, nw1, nb1, nw2, nb2, gw1, gb1, gw2, gb2):
    n, din = x_in.shape
    grid = n // TN

    def body(x_ref, u_ref, b_ref, nw1r, nb1r, nw2r, nb2r,
             gw1r, gb1r, gw2r, gb2r, xt_ref, xtb_ref, ubb_ref):
        xh = _mlp2(x_ref[...], nw1r[...], nb1r[...], nw2r[...], nb2r[...])
        u_head = _mlp2(u_ref[...], gw1r[...], gb1r[...], gw2r[...], gb2r[...])
        oh = (b_ref[...] == lax.broadcasted_iota(I32, (1, 64), 1)).astype(F32)
        xt_ref[...] = xh
        xtb_ref[...] = xh.astype(BF16)
        ubb_ref[...] = _dot(oh, u_head).astype(BF16)

    cst = lambda *_: (0, 0)
    out32 = jax.ShapeDtypeStruct((n, 32), F32)
    return pl.pallas_call(
        body,
        grid=(grid,),
        in_specs=[
            pl.BlockSpec((TN, din), lambda i: (i, 0)),
            pl.BlockSpec((64, 32), cst),
            pl.BlockSpec((TN, 1), lambda i: (i, 0)),
            pl.BlockSpec(nw1.shape, cst), pl.BlockSpec(nb1.shape, cst),
            pl.BlockSpec(nw2.shape, cst), pl.BlockSpec(nb2.shape, cst),
            pl.BlockSpec(gw1.shape, cst), pl.BlockSpec(gb1.shape, cst),
            pl.BlockSpec(gw2.shape, cst), pl.BlockSpec(gb2.shape, cst),
        ],
        out_specs=[pl.BlockSpec((TN, 32), lambda i: (i, 0)),
                   pl.BlockSpec((TN, 32), lambda i: (i, 0)),
                   pl.BlockSpec((TN, 32), lambda i: (i, 0))],
        out_shape=[out32, jax.ShapeDtypeStruct((n, 32), BF16),
                   jax.ShapeDtypeStruct((n, 32), BF16)],
    )(x_in, u_src, batch_col, nw1, nb1, nw2, nb2, gw1, gb1, gw2, gb2)


def _tc_edge(e_src, gxr, gub, gxc, ew1, eb1, ew2, eb2,
             we1, be1, we2, be2, we3, be3, block0):
    e, din = e_src.shape
    grid = e // TE

    def body(es_ref, xr_ref, ub_ref, xc_ref, ew1r, eb1r, ew2r, eb2r,
             we1r, be1r, we2r, be2r, we3r, be3r,
             eh_ref, eo_ref):
        e0 = _mlp2(es_ref[...], ew1r[...], eb1r[...], ew2r[...], eb2r[...])
        ein = jnp.concatenate(
            [xr_ref[...], xc_ref[...], e0.astype(BF16), ub_ref[...]], axis=1)
        z = _relu(_dot(ein, we1r[...]) + be1r[...])
        z = _relu(_dot(z, we2r[...]) + be2r[...])
        eh = _dot(z, we3r[...]) + be3r[...]
        eh_ref[...] = eh
        eo_ref[...] = (e0 if block0 else es_ref[...]) + eh

    cst = lambda *_: (0, 0)
    out32 = jax.ShapeDtypeStruct((e, 32), F32)
    ed = pl.BlockSpec((TE, 32), lambda i: (i, 0))
    return pl.pallas_call(
        body,
        grid=(grid,),
        in_specs=[
            pl.BlockSpec((TE, din), lambda i: (i, 0)), ed, ed, ed,
            pl.BlockSpec(ew1.shape, cst), pl.BlockSpec(eb1.shape, cst),
            pl.BlockSpec(ew2.shape, cst), pl.BlockSpec(eb2.shape, cst),
            pl.BlockSpec(we1.shape, cst), pl.BlockSpec(be1.shape, cst),
            pl.BlockSpec(we2.shape, cst), pl.BlockSpec(be2.shape, cst),
            pl.BlockSpec(we3.shape, cst), pl.BlockSpec(be3.shape, cst),
        ],
        out_specs=[ed, ed],
        out_shape=[out32, out32],
    )(e_src, gxr, gub, gxc, ew1, eb1, ew2, eb2,
      we1, be1, we2, be2, we3, be3)


def _tc_node_global(s0, s1, c0, c1, xtab, ubtab, x_res, u_src, batch_r3,
                    wn1, bn1, wn2, bn2, wn3, bn3,
                    hw1, hb1, hw2, hb2,
                    gwa, gwb, gwc, gb1, gw2, gb2, gw3, gb3,
                    block0):
    n = xtab.shape[0]
    grid = n // TN

    def body(s0_ref, s1_ref, c0_ref, c1_ref, xt_ref, ub_ref, xr_ref, u_ref,
             br_ref, wn1r, bn1r, wn2r, bn2r, wn3r, bn3r,
             hw1r, hb1r, hw2r, hb2r,
             gwar, gwbr, gwcr, gb1r, gw2r, gb2r, gw3r, gb3r,
             xo_ref, uo_ref, gn_acc, ge_acc, cn_acc, ce_acc):
        i = pl.program_id(0)

        @pl.when(i == 0)
        def _():
            gn_acc[...] = jnp.zeros_like(gn_acc)
            ge_acc[...] = jnp.zeros_like(ge_acc)
            cn_acc[...] = jnp.zeros_like(cn_acc)
            ce_acc[...] = jnp.zeros_like(ce_acc)

        deg = c0_ref[...] + c1_ref[...]
        s = s0_ref[...] + s1_ref[...]
        agg = s * (1.0 / jnp.maximum(deg[:, 0:1], 1.0))
        xh = xt_ref[...]
        ub = ub_ref[...].astype(F32)
        nin = jnp.concatenate([agg, xh, ub], axis=1)
        h = _relu(_dot(nin, wn1r[...]) + bn1r[...])
        h = _relu(_dot(h, wn2r[...]) + bn2r[...])
        xh_new = _dot(h, wn3r[...]) + bn3r[...]
        xo_ref[...] = xr_ref[...] + xh_new

        oht = (lax.broadcasted_iota(I32, (64, 1), 0)
               == br_ref[0]).astype(F32)
        gn_acc[...] += _dot(oht, xh_new)
        ge_acc[...] += _dot(oht, s)
        cn_acc[...] += _dot(oht, jnp.ones((TN, 8), F32))
        ce_acc[...] += _dot(oht, deg[:, 0:8])

        @pl.when(i == grid - 1)
        def _():
            u_head = _mlp2(u_ref[...], hw1r[...], hb1r[...],
                           hw2r[...], hb2r[...])
            node_mean = gn_acc[...] / jnp.maximum(cn_acc[...][:, 0:1], 1.0)
            edge_mean = ge_acc[...] / jnp.maximum(ce_acc[...][:, 0:1], 1.0)
            g = _relu(_dot(u_head, gwar[...]) + _dot(node_mean, gwbr[...])
                      + _dot(edge_mean, gwcr[...]) + gb1r[...])
            g = _relu(_dot(g, gw2r[...]) + gb2r[...])
            uh_new = _dot(g, gw3r[...]) + gb3r[...]
            u_base = u_head if block0 else u_ref[...]
            uo_ref[...] = u_base + uh_new

    cst = lambda *_: (0, 0)
    nd32 = pl.BlockSpec((TN, 32), lambda i: (i, 0))
    nd16 = pl.BlockSpec((TN, 16), lambda i: (i, 0))
    return pl.pallas_call(
        body,
        grid=(grid,),
        in_specs=[
            nd32, nd32, nd16, nd16, nd32, nd32, nd32,
            pl.BlockSpec((64, 32), cst),
            pl.BlockSpec((1, 1, TN), lambda i: (i, 0, 0)),
            pl.BlockSpec(wn1.shape, cst), pl.BlockSpec(bn1.shape, cst),
            pl.BlockSpec(wn2.shape, cst), pl.BlockSpec(bn2.shape, cst),
            pl.BlockSpec(wn3.shape, cst), pl.BlockSpec(bn3.shape, cst),
            pl.BlockSpec(hw1.shape, cst), pl.BlockSpec(hb1.shape, cst),
            pl.BlockSpec(hw2.shape, cst), pl.BlockSpec(hb2.shape, cst),
            pl.BlockSpec(gwa.shape, cst), pl.BlockSpec(gwb.shape, cst),
            pl.BlockSpec(gwc.shape, cst), pl.BlockSpec(gb1.shape, cst),
            pl.BlockSpec(gw2.shape, cst), pl.BlockSpec(gb2.shape, cst),
            pl.BlockSpec(gw3.shape, cst), pl.BlockSpec(gb3.shape, cst),
        ],
        out_specs=[nd32, pl.BlockSpec((64, 32), cst)],
        out_shape=[jax.ShapeDtypeStruct((n, 32), F32),
                   jax.ShapeDtypeStruct((64, 32), F32)],
        scratch_shapes=[pltpu.VMEM((64, 32), F32), pltpu.VMEM((64, 32), F32),
                        pltpu.VMEM((64, 8), F32), pltpu.VMEM((64, 8), F32)],
    )(s0, s1, c0, c1, xtab, ubtab, x_res, u_src, batch_r3,
      wn1, bn1, wn2, bn2, wn3, bn3, hw1, hb1, hw2, hb2,
      gwa, gwb, gwc, gb1, gw2, gb2, gw3, gb3)


def _lin(layer):
    w, b = layer
    return w, b.reshape(1, -1)


def _run(x, edge_index, edge_attr, u, batch, params):
    n = x.shape[0]
    e = edge_index.shape[1]
    row = edge_index[0].astype(I32)
    col = edge_index[1].astype(I32)
    batch32 = batch.astype(I32)
    batch_col = batch32[:, None]
    batch_r3 = batch32.reshape(n // TN, 1, TN)
    zeros32 = jnp.zeros((n, 32), F32)
    zeros16 = jnp.zeros((n, 16), F32)
    ones16 = jnp.ones((CH, 16), F32)

    counts = _sc_counts(row, n, zeros16, ones16)
    c0, c1 = counts[0], counts[1]

    nfw1, nfb1 = _lin(params['node_dense_first'][0])
    nfw2, nfb2 = _lin(params['node_dense_first'][1])
    gfw1, gfb1 = _lin(params['global_dense_first'][0])
    gfw2, gfb2 = _lin(params['global_dense_first'][1])
    efw1, efb1 = _lin(params['edge_dense_first'][0])
    efw2, efb2 = _lin(params['edge_dense_first'][1])

    x_out = None
    edge_out = None
    u_out = None
    for i in range(3):
        mp = params['megnet'][i]
        if i == 0:
            x_in, u_src, e_src = x, u, edge_attr
            nw = (nfw1, nfb1, nfw2, nfb2)
            gw = (gfw1, gfb1, gfw2, gfb2)
            ew = (efw1, efb1, efw2, efb2)
        else:
            x_in, u_src, e_src = x_out, u_out, edge_out
            nd1, nd2 = params['node_dense'][i - 1]
            gd1, gd2 = params['global_dense'][i - 1]
            ed1, ed2 = params['edge_dense'][i - 1]
            nw = _lin(nd1) + _lin(nd2)
            gw = _lin(gd1) + _lin(gd2)
            ew = _lin(ed1) + _lin(ed2)

        xtab, xtab_b, ubtab_b = _tc_prep(x_in, u_src, batch_col, *nw, *gw)
        gxr, gub, gxc = _sc_gather(xtab_b, ubtab_b, row, col)

        ew1m, eb1m = _lin(mp['edge_mlp'][0])
        ew2m, eb2m = _lin(mp['edge_mlp'][1])
        ew3m, eb3m = _lin(mp['edge_mlp'][2])
        eh, edge_out = _tc_edge(e_src, gxr, gub, gxc, *ew,
                                ew1m.astype(BF16), eb1m, ew2m, eb2m,
                                ew3m, eb3m, block0=(i == 0))

        seg = _sc_scatter_add(eh, row, n, zeros32)
        s0, s1 = seg[0], seg[1]

        nw1m, nb1m = _lin(mp['node_mlp'][0])
        nw2m, nb2m = _lin(mp['node_mlp'][1])
        nw3m, nb3m = _lin(mp['node_mlp'][2])
        gw1m, gb1m = _lin(mp['global_mlp'][0])
        gw2m, gb2m = _lin(mp['global_mlp'][1])
        gw3m, gb3m = _lin(mp['global_mlp'][2])
        gwa, gwb, gwc = gw1m[0:32], gw1m[32:64], gw1m[64:96]

        x_res = xtab if i == 0 else x_in
        x_out, u_out = _tc_node_global(
            s0, s1, c0, c1, xtab, ubtab_b, x_res, u_src, batch_r3,
            nw1m, nb1m, nw2m, nb2m, nw3m, nb3m,
            *gw, gwa, gwb, gwc, gb1m, gw2m, gb2m, gw3m, gb3m,
            block0=(i == 0))

    return (x_out, edge_out, u_out)


_run_jit = jax.jit(_run)


def kernel(x, edge_index, edge_attr, u, batch, params):
    return _run_jit(x, edge_index, edge_attr, u, batch, params)


# speedup vs baseline: 6.4839x; 1.7170x over previous
import functools

import jax
import jax.numpy as jnp
from jax import lax
from jax.experimental import pallas as pl
from jax.experimental.pallas import tpu as pltpu
from jax.experimental.pallas import tpu_sc as plsc

F32 = jnp.float32
I32 = jnp.int32

NC = 2
NS = 16
NW = NC * NS

TN = 2000
TE = 8000
CH = 1000

_SC_PARAMS = pltpu.CompilerParams(use_tc_tiling_on_sc=False)
_SC_PARAMS_BIG = pltpu.CompilerParams(use_tc_tiling_on_sc=False,
                                      internal_scratch_in_bytes=0)


def _relu(v):
    return jnp.maximum(v, 0.0)


def _dot(a, b):
    return jnp.dot(a, b, preferred_element_type=F32)


def _mlp2(v, w1, b1, w2, b2):
    return _dot(_relu(_dot(v, w1) + b1), w2) + b2


def _sc_gather(atab, ctab, row, col):
    e = row.shape[0]
    ew = e // NW
    nch = ew // CH
    mesh = plsc.VectorSubcoreMesh(core_axis_name="c", subcore_axis_name="s")
    out_t = jax.ShapeDtypeStruct((e, 64), F32)

    @functools.partial(
        pl.kernel, mesh=mesh,
        compiler_params=_SC_PARAMS,
        out_type=(out_t, out_t),
        scratch_types=[
            pltpu.VMEM((CH,), I32), pltpu.VMEM((CH,), I32),
            pltpu.VMEM((CH, 64), F32), pltpu.SemaphoreType.DMA,
        ],
    )
    def k(at_hbm, ct_hbm, row_hbm, col_hbm, ga_hbm, gc_hbm,
          idxr, idxc, buf, sem):
        wid = lax.axis_index("s") * NC + lax.axis_index("c")
        base = wid * ew

        @pl.loop(0, nch)
        def _(kk):
            off = base + kk * CH
            pltpu.sync_copy(row_hbm.at[pl.ds(off, CH)], idxr)
            pltpu.sync_copy(col_hbm.at[pl.ds(off, CH)], idxc)
            pltpu.async_copy(at_hbm.at[idxr], buf, sem).wait()
            pltpu.sync_copy(buf, ga_hbm.at[pl.ds(off, CH)])
            pltpu.async_copy(ct_hbm.at[idxc], buf, sem).wait()
            pltpu.sync_copy(buf, gc_hbm.at[pl.ds(off, CH)])

    return k(atab, ctab, row, col)


def _sc_scatter_add(vals, row, n, zeros32):
    e = vals.shape[0]
    ew = e // NW
    chs = 200
    nch = ew // chs
    nps = n // NS
    mesh = plsc.VectorSubcoreMesh(core_axis_name="c", subcore_axis_name="s")

    @functools.partial(
        pl.kernel, mesh=mesh,
        compiler_params=_SC_PARAMS_BIG,
        out_type=jax.ShapeDtypeStruct((NC, n, 32), F32),
        scratch_types=[
            pltpu.VMEM((chs,), I32), pltpu.VMEM((chs, 32), F32),
            pltpu.VMEM_SHARED((n, 32), F32),
        ],
    )
    def k(vals_hbm, row_hbm, z_hbm, out_hbm, idx_v, val_v, shared):
        cid = lax.axis_index("c")
        sid = lax.axis_index("s")
        pltpu.sync_copy(z_hbm.at[pl.ds(sid * nps, nps)],
                        shared.at[pl.ds(sid * nps, nps)])
        plsc.subcore_barrier()
        base = cid * (ew * NS) + sid * ew

        @pl.loop(0, nch)
        def _(kk):
            off = base + kk * chs
            pltpu.sync_copy(row_hbm.at[pl.ds(off, chs)], idx_v)
            pltpu.sync_copy(vals_hbm.at[pl.ds(off, chs)], val_v)
            pltpu.sync_copy(val_v, shared.at[idx_v], add=True)

        plsc.subcore_barrier()
        pltpu.sync_copy(shared.at[pl.ds(sid * nps, nps)],
                        out_hbm.at[cid, pl.ds(sid * nps, nps)])

    return k(vals, row, zeros32)


def _sc_counts(row, n, zeros16, ones16):
    e = row.shape[0]
    ew = e // NW
    nch = ew // CH
    nps = n // NS
    mesh = plsc.VectorSubcoreMesh(core_axis_name="c", subcore_axis_name="s")

    @functools.partial(
        pl.kernel, mesh=mesh,
        compiler_params=_SC_PARAMS,
        out_type=jax.ShapeDtypeStruct((NC, n, 16), F32),
        scratch_types=[
            pltpu.VMEM((CH,), I32), pltpu.VMEM((CH, 16), F32),
            pltpu.VMEM_SHARED((n, 16), F32),
        ],
    )
    def k(row_hbm, z_hbm, ones_hbm, out_hbm, idx_v, ones_v, shared):
        cid = lax.axis_index("c")
        sid = lax.axis_index("s")
        pltpu.sync_copy(z_hbm.at[pl.ds(sid * nps, nps)],
                        shared.at[pl.ds(sid * nps, nps)])
        pltpu.sync_copy(ones_hbm, ones_v)
        plsc.subcore_barrier()
        base = cid * (ew * NS) + sid * ew

        @pl.loop(0, nch)
        def _(kk):
            off = base + kk * CH
            pltpu.sync_copy(row_hbm.at[pl.ds(off, CH)], idx_v)
            pltpu.sync_copy(ones_v, shared.at[idx_v], add=True)

        plsc.subcore_barrier()
        pltpu.sync_copy(shared.at[pl.ds(sid * nps, nps)],
                        out_hbm.at[cid, pl.ds(sid * nps, nps)])

    return k(row, zeros16, ones16)


def _tc_prep(x_in, u_src, batch_col, nw1, nb1, nw2, nb2, gw1, gb1, gw2, gb2,
             wxr, wxc, wub, be1):
    n, din = x_in.shape
    grid = n // TN

    def body(x_ref, u_ref, b_ref, nw1r, nb1r, nw2r, nb2r,
             gw1r, gb1r, gw2r, gb2r, wxrr, wxcr, wubr, be1r,
             xt_ref, ub_ref, at_ref, ct_ref):
        xh = _mlp2(x_ref[...], nw1r[...], nb1r[...], nw2r[...], nb2r[...])
        u_head = _mlp2(u_ref[...], gw1r[...], gb1r[...], gw2r[...], gb2r[...])
        oh = (b_ref[...] == lax.broadcasted_iota(I32, (1, 64), 1)).astype(F32)
        ub = _dot(oh, u_head)
        xt_ref[...] = xh
        ub_ref[...] = ub
        at_ref[...] = _dot(xh, wxrr[...]) + _dot(ub, wubr[...]) + be1r[...]
        ct_ref[...] = _dot(xh, wxcr[...])

    cst = lambda *_: (0, 0)
    out32 = jax.ShapeDtypeStruct((n, 32), F32)
    outp = jax.ShapeDtypeStruct((n, 64), F32)
    return pl.pallas_call(
        body,
        grid=(grid,),
        in_specs=[
            pl.BlockSpec((TN, din), lambda i: (i, 0)),
            pl.BlockSpec((64, 32), cst),
            pl.BlockSpec((TN, 1), lambda i: (i, 0)),
            pl.BlockSpec(nw1.shape, cst), pl.BlockSpec(nb1.shape, cst),
            pl.BlockSpec(nw2.shape, cst), pl.BlockSpec(nb2.shape, cst),
            pl.BlockSpec(gw1.shape, cst), pl.BlockSpec(gb1.shape, cst),
            pl.BlockSpec(gw2.shape, cst), pl.BlockSpec(gb2.shape, cst),
            pl.BlockSpec(wxr.shape, cst), pl.BlockSpec(wxc.shape, cst),
            pl.BlockSpec(wub.shape, cst), pl.BlockSpec(be1.shape, cst),
        ],
        out_specs=[pl.BlockSpec((TN, 32), lambda i: (i, 0)),
                   pl.BlockSpec((TN, 32), lambda i: (i, 0)),
                   pl.BlockSpec((TN, 64), lambda i: (i, 0)),
                   pl.BlockSpec((TN, 64), lambda i: (i, 0))],
        out_shape=[out32, out32, outp, outp],
    )(x_in, u_src, batch_col, nw1, nb1, nw2, nb2, gw1, gb1, gw2, gb2,
      wxr, wxc, wub, be1)


def _tc_edge(es2, ga2, gc2, ew1d, eb1d, ew2d, eb2d,
             wehd, we2dd, be2dd, we3dd, be3dd, block0):
    cst = lambda *_: (0, 0)
    if block0:
        e_, din = es2.shape
        eh_ = e_ // 2
        th = min(TE, e_) // 2
        grid = eh_ // th
        goff = grid

        def body(esa_ref, esb_ref, ga_ref, gc_ref, ew1r, eb1r, ew2r, eb2r,
                 wehr, we2dr, be2dr, we3dr, be3dr, eh_ref, eo_ref):
            e0a = _mlp2(esa_ref[...], ew1r[...], eb1r[...], ew2r[...],
                        eb2r[...])
            e0b = _mlp2(esb_ref[...], ew1r[...], eb1r[...], ew2r[...],
                        eb2r[...])
            e0 = jnp.concatenate([e0a, e0b], axis=1)
            z = _relu(ga_ref[...] + gc_ref[...] + _dot(e0, wehr[...]))
            z = _relu(_dot(z, we2dr[...]) + be2dr[...])
            ehp = _dot(z, we3dr[...]) + be3dr[...]
            eh_ref[...] = ehp
            eo_ref[...] = e0 + ehp

        in0 = [pl.BlockSpec((th, din), lambda i: (i, 0)),
               pl.BlockSpec((th, din), lambda i: (i + goff, 0))]
        args0 = (es2, es2)
    else:
        eh_, din2 = es2.shape
        th = min(TE, 2 * eh_) // 2
        grid = eh_ // th

        def body(es_ref, ga_ref, gc_ref, ew1r, eb1r, ew2r, eb2r,
                 wehr, we2dr, be2dr, we3dr, be3dr, eh_ref, eo_ref):
            es = es_ref[...]
            e0 = _mlp2(es, ew1r[...], eb1r[...], ew2r[...], eb2r[...])
            z = _relu(ga_ref[...] + gc_ref[...] + _dot(e0, wehr[...]))
            z = _relu(_dot(z, we2dr[...]) + be2dr[...])
            ehp = _dot(z, we3dr[...]) + be3dr[...]
            eh_ref[...] = ehp
            eo_ref[...] = es + ehp

        in0 = [pl.BlockSpec((th, din2), lambda i: (i, 0))]
        args0 = (es2,)

    pk = pl.BlockSpec((th, 128), lambda i: (i, 0))
    p64 = pl.BlockSpec((th, 64), lambda i: (i, 0))
    return pl.pallas_call(
        body,
        grid=(grid,),
        in_specs=in0 + [
            pk, pk,
            pl.BlockSpec(ew1d.shape, cst), pl.BlockSpec(eb1d.shape, cst),
            pl.BlockSpec(ew2d.shape, cst), pl.BlockSpec(eb2d.shape, cst),
            pl.BlockSpec(wehd.shape, cst),
            pl.BlockSpec(we2dd.shape, cst), pl.BlockSpec(be2dd.shape, cst),
            pl.BlockSpec(we3dd.shape, cst), pl.BlockSpec(be3dd.shape, cst),
        ],
        out_specs=[p64, p64],
        out_shape=[jax.ShapeDtypeStruct((eh_, 64), F32),
                   jax.ShapeDtypeStruct((eh_, 64), F32)],
    )(*args0, ga2, gc2, ew1d, eb1d, ew2d, eb2d, wehd, we2dd, be2dd,
      we3dd, be3dd)


def _tc_node_global(seg4, cnt8, xtab, ubtab, x_res, u_src, batch_r3,
                    wn1, bn1, wn2, bn2, wn3, bn3,
                    hw1, hb1, hw2, hb2,
                    gwa, gwb, gwc, gb1, gw2, gb2, gw3, gb3,
                    block0):
    n = xtab.shape[0]
    grid = n // TN

    def body(s0_ref, s1_ref, c0_ref, c1_ref, xt_ref, ub_ref, xr_ref, u_ref,
             br_ref, wn1r, bn1r, wn2r, bn2r, wn3r, bn3r,
             hw1r, hb1r, hw2r, hb2r,
             gwar, gwbr, gwcr, gb1r, gw2r, gb2r, gw3r, gb3r,
             xo_ref, uo_ref, gn_acc, ge_acc, cn_acc, ce_acc):
        i = pl.program_id(0)

        @pl.when(i == 0)
        def _():
            gn_acc[...] = jnp.zeros_like(gn_acc)
            ge_acc[...] = jnp.zeros_like(ge_acc)
            cn_acc[...] = jnp.zeros_like(cn_acc)
            ce_acc[...] = jnp.zeros_like(ce_acc)

        deg = c0_ref[0] + c1_ref[0]
        s = s0_ref[0] + s1_ref[0]
        agg = s * (1.0 / jnp.maximum(deg[:, 0:1], 1.0))
        xh = xt_ref[...]
        ub = ub_ref[...]
        nin = jnp.concatenate([agg, xh, ub], axis=1)
        h = _relu(_dot(nin, wn1r[...]) + bn1r[...])
        h = _relu(_dot(h, wn2r[...]) + bn2r[...])
        xh_new = _dot(h, wn3r[...]) + bn3r[...]
        xo_ref[...] = xr_ref[...] + xh_new

        oht = (lax.broadcasted_iota(I32, (64, 1), 0)
               == br_ref[0]).astype(F32)
        gn_acc[...] += _dot(oht, xh_new)
        ge_acc[...] += _dot(oht, s)
        cn_acc[...] += _dot(oht, jnp.ones((TN, 8), F32))
        ce_acc[...] += _dot(oht, deg[:, 0:8])

        @pl.when(i == grid - 1)
        def _():
            u_head = _mlp2(u_ref[...], hw1r[...], hb1r[...],
                           hw2r[...], hb2r[...])
            node_mean = gn_acc[...] / jnp.maximum(cn_acc[...][:, 0:1], 1.0)
            edge_mean = ge_acc[...] / jnp.maximum(ce_acc[...][:, 0:1], 1.0)
            g = _relu(_dot(u_head, gwar[...]) + _dot(node_mean, gwbr[...])
                      + _dot(edge_mean, gwcr[...]) + gb1r[...])
            g = _relu(_dot(g, gw2r[...]) + gb2r[...])
            uh_new = _dot(g, gw3r[...]) + gb3r[...]
            u_base = u_head if block0 else u_ref[...]
            uo_ref[...] = u_base + uh_new

    cst = lambda *_: (0, 0)
    nd32 = pl.BlockSpec((TN, 32), lambda i: (i, 0))
    sspec0 = pl.BlockSpec((1, TN, 32), lambda i: (0, i, 0))
    sspec1 = pl.BlockSpec((1, TN, 32), lambda i: (1, i, 0))
    cspec0 = pl.BlockSpec((1, TN, 16), lambda i: (0, i, 0))
    cspec1 = pl.BlockSpec((1, TN, 16), lambda i: (1, i, 0))
    return pl.pallas_call(
        body,
        grid=(grid,),
        in_specs=[
            sspec0, sspec1, cspec0, cspec1, nd32, nd32, nd32,
            pl.BlockSpec((64, 32), cst),
            pl.BlockSpec((1, 1, TN), lambda i: (i, 0, 0)),
            pl.BlockSpec(wn1.shape, cst), pl.BlockSpec(bn1.shape, cst),
            pl.BlockSpec(wn2.shape, cst), pl.BlockSpec(bn2.shape, cst),
            pl.BlockSpec(wn3.shape, cst), pl.BlockSpec(bn3.shape, cst),
            pl.BlockSpec(hw1.shape, cst), pl.BlockSpec(hb1.shape, cst),
            pl.BlockSpec(hw2.shape, cst), pl.BlockSpec(hb2.shape, cst),
            pl.BlockSpec(gwa.shape, cst), pl.BlockSpec(gwb.shape, cst),
            pl.BlockSpec(gwc.shape, cst), pl.BlockSpec(gb1.shape, cst),
            pl.BlockSpec(gw2.shape, cst), pl.BlockSpec(gb2.shape, cst),
            pl.BlockSpec(gw3.shape, cst), pl.BlockSpec(gb3.shape, cst),
        ],
        out_specs=[nd32, pl.BlockSpec((64, 32), cst)],
        out_shape=[jax.ShapeDtypeStruct((n, 32), F32),
                   jax.ShapeDtypeStruct((64, 32), F32)],
        scratch_shapes=[pltpu.VMEM((64, 32), F32), pltpu.VMEM((64, 32), F32),
                        pltpu.VMEM((64, 8), F32), pltpu.VMEM((64, 8), F32)],
    )(seg4, seg4, cnt8, cnt8, xtab, ubtab, x_res, u_src, batch_r3,
      wn1, bn1, wn2, bn2, wn3, bn3, hw1, hb1, hw2, hb2,
      gwa, gwb, gwc, gb1, gw2, gb2, gw3, gb3)


def _lin(layer):
    w, b = layer
    return w, b.reshape(1, -1)


def _diag2(w):
    dk, dn = w.shape
    z = jnp.zeros((dk, dn), F32)
    return jnp.concatenate([
        jnp.concatenate([w, z], axis=1),
        jnp.concatenate([z, w], axis=1),
    ], axis=0)


def _dup2(b):
    return jnp.concatenate([b, b], axis=1)


def _run(x, edge_index, edge_attr, u, batch, params):
    n = x.shape[0]
    e = edge_index.shape[1]
    row = edge_index[0].astype(I32)
    col = edge_index[1].astype(I32)
    row = jnp.stack([row[:e // 2], row[e // 2:]], axis=1).reshape(e)
    col = jnp.stack([col[:e // 2], col[e // 2:]], axis=1).reshape(e)
    batch32 = batch.astype(I32)
    batch_col = batch32[:, None]
    batch_r3 = batch32.reshape(n // TN, 1, TN)
    zeros32 = jnp.zeros((n, 32), F32)
    zeros16 = jnp.zeros((n, 16), F32)
    ones16 = jnp.ones((CH, 16), F32)

    cnt8 = _sc_counts(row, n, zeros16, ones16)

    nfw1, nfb1 = _lin(params['node_dense_first'][0])
    nfw2, nfb2 = _lin(params['node_dense_first'][1])
    gfw1, gfb1 = _lin(params['global_dense_first'][0])
    gfw2, gfb2 = _lin(params['global_dense_first'][1])
    efw1, efb1 = _lin(params['edge_dense_first'][0])
    efw2, efb2 = _lin(params['edge_dense_first'][1])

    x_out = None
    edge_out = None
    u_out = None
    for i in range(3):
        mp = params['megnet'][i]
        if i == 0:
            x_in, u_src, e_src = x, u, edge_attr
            nw = (nfw1, nfb1, nfw2, nfb2)
            gw = (gfw1, gfb1, gfw2, gfb2)
            ew = (efw1, efb1, efw2, efb2)
        else:
            x_in, u_src, e_src = x_out, u_out, edge_out
            nd1, nd2 = params['node_dense'][i - 1]
            gd1, gd2 = params['global_dense'][i - 1]
            ed1, ed2 = params['edge_dense'][i - 1]
            nw = _lin(nd1) + _lin(nd2)
            gw = _lin(gd1) + _lin(gd2)
            ew = _lin(ed1) + _lin(ed2)

        ew1m, eb1m = _lin(mp['edge_mlp'][0])
        ew2m, eb2m = _lin(mp['edge_mlp'][1])
        ew3m, eb3m = _lin(mp['edge_mlp'][2])
        wxr, wxc, weh, wub = (ew1m[0:32], ew1m[32:64], ew1m[64:96],
                              ew1m[96:128])

        xtab, ubtab, atab, ctab = _tc_prep(
            x_in, u_src, batch_col, *nw, *gw, wxr, wxc, wub, eb1m)
        ga, gc = _sc_gather(atab, ctab, row, col)

        ew1, eb1, ew2, eb2 = ew
        if i == 0:
            mini = (ew1, eb1, ew2, eb2)
        else:
            mini = (_diag2(ew1), _dup2(eb1), _diag2(ew2), _dup2(eb2))
        eh2, edge_out = _tc_edge(
            e_src, ga.reshape(e // 2, 128), gc.reshape(e // 2, 128),
            *mini,
            _diag2(weh), _diag2(ew2m), _dup2(eb2m), _diag2(ew3m),
            _dup2(eb3m), block0=(i == 0))

        seg4 = _sc_scatter_add(eh2.reshape(e, 32), row, n, zeros32)

        nw1m, nb1m = _lin(mp['node_mlp'][0])
        nw2m, nb2m = _lin(mp['node_mlp'][1])
        nw3m, nb3m = _lin(mp['node_mlp'][2])
        gw1m, gb1m = _lin(mp['global_mlp'][0])
        gw2m, gb2m = _lin(mp['global_mlp'][1])
        gw3m, gb3m = _lin(mp['global_mlp'][2])
        gwa, gwb, gwc = gw1m[0:32], gw1m[32:64], gw1m[64:96]

        x_res = xtab if i == 0 else x_in
        x_out, u_out = _tc_node_global(
            seg4, cnt8, xtab, ubtab, x_res, u_src, batch_r3,
            nw1m, nb1m, nw2m, nb2m, nw3m, nb3m,
            *gw, gwa, gwb, gwc, gb1m, gw2m, gb2m, gw3m, gb3m,
            block0=(i == 0))

    eo = edge_out.reshape(e // 2, 2, 32).transpose(1, 0, 2).reshape(e, 32)
    return (x_out, eo, u_out)


_run_jit = jax.jit(_run)


def kernel(x, edge_index, edge_attr, u, batch, params):
    return _run_jit(x, edge_index, edge_attr, u, batch, params)


# speedup vs baseline: 7.4509x; 1.1491x over previous
import functools

import jax
import jax.numpy as jnp
from jax import lax
from jax.experimental import pallas as pl
from jax.experimental.pallas import tpu as pltpu
from jax.experimental.pallas import tpu_sc as plsc

F32 = jnp.float32
I32 = jnp.int32

NC = 2
NS = 16
NW = NC * NS

TN = 2000
TE = 8000
CH = 1000

_SC_PARAMS = pltpu.CompilerParams(use_tc_tiling_on_sc=False)
_SC_PARAMS_BIG = pltpu.CompilerParams(use_tc_tiling_on_sc=False,
                                      internal_scratch_in_bytes=0)


def _relu(v):
    return jnp.maximum(v, 0.0)


def _dot(a, b):
    return jnp.dot(a, b, preferred_element_type=F32)


def _mlp2(v, w1, b1, w2, b2):
    return _dot(_relu(_dot(v, w1) + b1), w2) + b2


def _sc_gather(atab, ctab, row, col):
    e = row.shape[0]
    ew = e // NW
    chg = 200
    nch = ew // chg
    npairs = (nch - 1) // 2
    mesh = plsc.VectorSubcoreMesh(core_axis_name="c", subcore_axis_name="s")
    out_t = jax.ShapeDtypeStruct((e, 64), F32)

    @functools.partial(
        pl.kernel, mesh=mesh,
        compiler_params=_SC_PARAMS,
        out_type=(out_t, out_t),
        scratch_types=[
            pltpu.VMEM((chg,), I32), pltpu.VMEM((chg,), I32),
            pltpu.VMEM((chg,), I32), pltpu.VMEM((chg,), I32),
            pltpu.VMEM((chg, 64), F32), pltpu.VMEM((chg, 64), F32),
            pltpu.VMEM((chg, 64), F32), pltpu.VMEM((chg, 64), F32),
            pltpu.SemaphoreType.DMA, pltpu.SemaphoreType.DMA,
        ],
    )
    def k(at_hbm, ct_hbm, row_hbm, col_hbm, ga_hbm, gc_hbm,
          ir0, ir1, ic0, ic1, ba0, ba1, bc0, bc1, sem0, sem1):
        wid = lax.axis_index("s") * NC + lax.axis_index("c")
        base = wid * ew
        irs = (ir0, ir1)
        ics = (ic0, ic1)
        bas = (ba0, ba1)
        bcs = (bc0, bc1)
        sems = (sem0, sem1)

        def load_and_fire(off, s):
            pltpu.sync_copy(row_hbm.at[pl.ds(off, chg)], irs[s])
            pltpu.sync_copy(col_hbm.at[pl.ds(off, chg)], ics[s])
            pltpu.async_copy(at_hbm.at[irs[s]], bas[s], sems[s])
            pltpu.async_copy(ct_hbm.at[ics[s]], bcs[s], sems[s])

        def drain_and_store(off, s):
            pltpu.make_async_copy(at_hbm.at[irs[s]], bas[s], sems[s]).wait()
            pltpu.make_async_copy(ct_hbm.at[ics[s]], bcs[s], sems[s]).wait()
            pltpu.sync_copy(bas[s], ga_hbm.at[pl.ds(off, chg)])
            pltpu.sync_copy(bcs[s], gc_hbm.at[pl.ds(off, chg)])

        load_and_fire(base, 0)

        @pl.loop(0, npairs)
        def _(j):
            c0 = base + (2 * j) * chg
            load_and_fire(c0 + chg, 1)
            drain_and_store(c0, 0)
            load_and_fire(c0 + 2 * chg, 0)
            drain_and_store(c0 + chg, 1)

        drain_and_store(base + (nch - 1) * chg, 0)

    return k(atab, ctab, row, col)


def _sc_scatter_add(vals, row, n, zeros32):
    e = vals.shape[0]
    ew = e // NW
    chs = 200
    nch = ew // chs
    npairs = (nch - 1) // 2
    nps = n // NS
    mesh = plsc.VectorSubcoreMesh(core_axis_name="c", subcore_axis_name="s")

    @functools.partial(
        pl.kernel, mesh=mesh,
        compiler_params=_SC_PARAMS_BIG,
        out_type=jax.ShapeDtypeStruct((NC, n, 32), F32),
        scratch_types=[
            pltpu.VMEM((chs,), I32), pltpu.VMEM((chs,), I32),
            pltpu.VMEM((chs, 32), F32), pltpu.VMEM((chs, 32), F32),
            pltpu.VMEM_SHARED((n, 32), F32),
            pltpu.SemaphoreType.DMA, pltpu.SemaphoreType.DMA,
        ],
    )
    def k(vals_hbm, row_hbm, z_hbm, out_hbm, ix0, ix1, va0, va1, shared,
          sem0, sem1):
        cid = lax.axis_index("c")
        sid = lax.axis_index("s")
        pltpu.sync_copy(z_hbm.at[pl.ds(sid * nps, nps)],
                        shared.at[pl.ds(sid * nps, nps)])
        plsc.subcore_barrier()
        base = cid * (ew * NS) + sid * ew
        ixs = (ix0, ix1)
        vas = (va0, va1)
        sems = (sem0, sem1)

        def fire(off, s):
            pltpu.async_copy(row_hbm.at[pl.ds(off, chs)], ixs[s], sems[s])
            pltpu.async_copy(vals_hbm.at[pl.ds(off, chs)], vas[s], sems[s])

        def scat(off, s):
            pltpu.make_async_copy(row_hbm.at[pl.ds(off, chs)], ixs[s],
                                  sems[s]).wait()
            pltpu.make_async_copy(vals_hbm.at[pl.ds(off, chs)], vas[s],
                                  sems[s]).wait()
            pltpu.sync_copy(vas[s], shared.at[ixs[s]], add=True)

        fire(base, 0)

        @pl.loop(0, npairs)
        def _(j):
            c0 = base + (2 * j) * chs
            fire(c0 + chs, 1)
            scat(c0, 0)
            fire(c0 + 2 * chs, 0)
            scat(c0 + chs, 1)

        scat(base + (nch - 1) * chs, 0)
        plsc.subcore_barrier()
        pltpu.sync_copy(shared.at[pl.ds(sid * nps, nps)],
                        out_hbm.at[cid, pl.ds(sid * nps, nps)])

    return k(vals, row, zeros32)


def _sc_counts(row, n, zeros16, ones16):
    e = row.shape[0]
    ew = e // NW
    nch = ew // CH
    nps = n // NS
    mesh = plsc.VectorSubcoreMesh(core_axis_name="c", subcore_axis_name="s")

    @functools.partial(
        pl.kernel, mesh=mesh,
        compiler_params=_SC_PARAMS,
        out_type=jax.ShapeDtypeStruct((NC, n, 16), F32),
        scratch_types=[
            pltpu.VMEM((CH,), I32), pltpu.VMEM((CH, 16), F32),
            pltpu.VMEM_SHARED((n, 16), F32),
        ],
    )
    def k(row_hbm, z_hbm, ones_hbm, out_hbm, idx_v, ones_v, shared):
        cid = lax.axis_index("c")
        sid = lax.axis_index("s")
        pltpu.sync_copy(z_hbm.at[pl.ds(sid * nps, nps)],
                        shared.at[pl.ds(sid * nps, nps)])
        pltpu.sync_copy(ones_hbm, ones_v)
        plsc.subcore_barrier()
        base = cid * (ew * NS) + sid * ew

        @pl.loop(0, nch)
        def _(kk):
            off = base + kk * CH
            pltpu.sync_copy(row_hbm.at[pl.ds(off, CH)], idx_v)
            pltpu.sync_copy(ones_v, shared.at[idx_v], add=True)

        plsc.subcore_barrier()
        pltpu.sync_copy(shared.at[pl.ds(sid * nps, nps)],
                        out_hbm.at[cid, pl.ds(sid * nps, nps)])

    return k(row, zeros16, ones16)


def _tc_prep(x_in, u_src, batch_col, nw1, nb1, nw2, nb2, gw1, gb1, gw2, gb2,
             wxr, wxc, wub, be1):
    n, din = x_in.shape
    grid = n // TN

    def body(x_ref, u_ref, b_ref, nw1r, nb1r, nw2r, nb2r,
             gw1r, gb1r, gw2r, gb2r, wxrr, wxcr, wubr, be1r,
             xt_ref, ub_ref, at_ref, ct_ref):
        xh = _mlp2(x_ref[...], nw1r[...], nb1r[...], nw2r[...], nb2r[...])
        u_head = _mlp2(u_ref[...], gw1r[...], gb1r[...], gw2r[...], gb2r[...])
        oh = (b_ref[...] == lax.broadcasted_iota(I32, (1, 64), 1)).astype(F32)
        ub = _dot(oh, u_head)
        xt_ref[...] = xh
        ub_ref[...] = ub
        at_ref[...] = _dot(xh, wxrr[...]) + _dot(ub, wubr[...]) + be1r[...]
        ct_ref[...] = _dot(xh, wxcr[...])

    cst = lambda *_: (0, 0)
    out32 = jax.ShapeDtypeStruct((n, 32), F32)
    outp = jax.ShapeDtypeStruct((n, 64), F32)
    return pl.pallas_call(
        body,
        grid=(grid,),
        in_specs=[
            pl.BlockSpec((TN, din), lambda i: (i, 0)),
            pl.BlockSpec((64, 32), cst),
            pl.BlockSpec((TN, 1), lambda i: (i, 0)),
            pl.BlockSpec(nw1.shape, cst), pl.BlockSpec(nb1.shape, cst),
            pl.BlockSpec(nw2.shape, cst), pl.BlockSpec(nb2.shape, cst),
            pl.BlockSpec(gw1.shape, cst), pl.BlockSpec(gb1.shape, cst),
            pl.BlockSpec(gw2.shape, cst), pl.BlockSpec(gb2.shape, cst),
            pl.BlockSpec(wxr.shape, cst), pl.BlockSpec(wxc.shape, cst),
            pl.BlockSpec(wub.shape, cst), pl.BlockSpec(be1.shape, cst),
        ],
        out_specs=[pl.BlockSpec((TN, 32), lambda i: (i, 0)),
                   pl.BlockSpec((TN, 32), lambda i: (i, 0)),
                   pl.BlockSpec((TN, 64), lambda i: (i, 0)),
                   pl.BlockSpec((TN, 64), lambda i: (i, 0))],
        out_shape=[out32, out32, outp, outp],
    )(x_in, u_src, batch_col, nw1, nb1, nw2, nb2, gw1, gb1, gw2, gb2,
      wxr, wxc, wub, be1)


def _tc_edge(es2, ga2, gc2, ew1d, eb1d, ew2d, eb2d,
             wehd, we2dd, be2dd, we3dd, be3dd, block0):
    eh_, din2 = es2.shape
    grid = 2 * eh_ // TE
    th = TE // 2

    def body(es_ref, ga_ref, gc_ref, ew1r, eb1r, ew2r, eb2r,
             wehr, we2dr, be2dr, we3dr, be3dr, eh_ref, eo_ref):
        es = es_ref[...]
        e0 = _mlp2(es, ew1r[...], eb1r[...], ew2r[...], eb2r[...])
        z = _relu(ga_ref[...] + gc_ref[...] + _dot(e0, wehr[...]))
        z = _relu(_dot(z, we2dr[...]) + be2dr[...])
        ehp = _dot(z, we3dr[...]) + be3dr[...]
        eh_ref[...] = ehp
        eo_ref[...] = (e0 if block0 else es) + ehp

    cst = lambda *_: (0, 0)
    pk = pl.BlockSpec((th, 128), lambda i: (i, 0))
    p64 = pl.BlockSpec((th, 64), lambda i: (i, 0))
    return pl.pallas_call(
        body,
        grid=(grid,),
        in_specs=[
            pl.BlockSpec((th, din2), lambda i: (i, 0)), pk, pk,
            pl.BlockSpec(ew1d.shape, cst), pl.BlockSpec(eb1d.shape, cst),
            pl.BlockSpec(ew2d.shape, cst), pl.BlockSpec(eb2d.shape, cst),
            pl.BlockSpec(wehd.shape, cst),
            pl.BlockSpec(we2dd.shape, cst), pl.BlockSpec(be2dd.shape, cst),
            pl.BlockSpec(we3dd.shape, cst), pl.BlockSpec(be3dd.shape, cst),
        ],
        out_specs=[p64, p64],
        out_shape=[jax.ShapeDtypeStruct((eh_, 64), F32),
                   jax.ShapeDtypeStruct((eh_, 64), F32)],
    )(es2, ga2, gc2, ew1d, eb1d, ew2d, eb2d, wehd, we2dd, be2dd,
      we3dd, be3dd)


def _tc_node_global(seg4, cnt8, xtab, ubtab, x_res, u_src, batch_r3,
                    wn1, bn1, wn2, bn2, wn3, bn3,
                    hw1, hb1, hw2, hb2,
                    gwa, gwb, gwc, gb1, gw2, gb2, gw3, gb3,
                    block0):
    n = xtab.shape[0]
    grid = n // TN

    def body(s0_ref, s1_ref, c0_ref, c1_ref, xt_ref, ub_ref, xr_ref, u_ref,
             br_ref, wn1r, bn1r, wn2r, bn2r, wn3r, bn3r,
             hw1r, hb1r, hw2r, hb2r,
             gwar, gwbr, gwcr, gb1r, gw2r, gb2r, gw3r, gb3r,
             xo_ref, uo_ref, gn_acc, ge_acc, cn_acc, ce_acc):
        i = pl.program_id(0)

        @pl.when(i == 0)
        def _():
            gn_acc[...] = jnp.zeros_like(gn_acc)
            ge_acc[...] = jnp.zeros_like(ge_acc)
            cn_acc[...] = jnp.zeros_like(cn_acc)
            ce_acc[...] = jnp.zeros_like(ce_acc)

        deg = c0_ref[0] + c1_ref[0]
        s = s0_ref[0] + s1_ref[0]
        agg = s * (1.0 / jnp.maximum(deg[:, 0:1], 1.0))
        xh = xt_ref[...]
        ub = ub_ref[...]
        nin = jnp.concatenate([agg, xh, ub], axis=1)
        h = _relu(_dot(nin, wn1r[...]) + bn1r[...])
        h = _relu(_dot(h, wn2r[...]) + bn2r[...])
        xh_new = _dot(h, wn3r[...]) + bn3r[...]
        xo_ref[...] = xr_ref[...] + xh_new

        oht = (lax.broadcasted_iota(I32, (64, 1), 0)
               == br_ref[0]).astype(F32)
        gn_acc[...] += _dot(oht, xh_new)
        ge_acc[...] += _dot(oht, s)
        cn_acc[...] += _dot(oht, jnp.ones((TN, 8), F32))
        ce_acc[...] += _dot(oht, deg[:, 0:8])

        @pl.when(i == grid - 1)
        def _():
            u_head = _mlp2(u_ref[...], hw1r[...], hb1r[...],
                           hw2r[...], hb2r[...])
            node_mean = gn_acc[...] / jnp.maximum(cn_acc[...][:, 0:1], 1.0)
            edge_mean = ge_acc[...] / jnp.maximum(ce_acc[...][:, 0:1], 1.0)
            g = _relu(_dot(u_head, gwar[...]) + _dot(node_mean, gwbr[...])
                      + _dot(edge_mean, gwcr[...]) + gb1r[...])
            g = _relu(_dot(g, gw2r[...]) + gb2r[...])
            uh_new = _dot(g, gw3r[...]) + gb3r[...]
            u_base = u_head if block0 else u_ref[...]
            uo_ref[...] = u_base + uh_new

    cst = lambda *_: (0, 0)
    nd32 = pl.BlockSpec((TN, 32), lambda i: (i, 0))
    sspec0 = pl.BlockSpec((1, TN, 32), lambda i: (0, i, 0))
    sspec1 = pl.BlockSpec((1, TN, 32), lambda i: (1, i, 0))
    cspec0 = pl.BlockSpec((1, TN, 16), lambda i: (0, i, 0))
    cspec1 = pl.BlockSpec((1, TN, 16), lambda i: (1, i, 0))
    return pl.pallas_call(
        body,
        grid=(grid,),
        in_specs=[
            sspec0, sspec1, cspec0, cspec1, nd32, nd32, nd32,
            pl.BlockSpec((64, 32), cst),
            pl.BlockSpec((1, 1, TN), lambda i: (i, 0, 0)),
            pl.BlockSpec(wn1.shape, cst), pl.BlockSpec(bn1.shape, cst),
            pl.BlockSpec(wn2.shape, cst), pl.BlockSpec(bn2.shape, cst),
            pl.BlockSpec(wn3.shape, cst), pl.BlockSpec(bn3.shape, cst),
            pl.BlockSpec(hw1.shape, cst), pl.BlockSpec(hb1.shape, cst),
            pl.BlockSpec(hw2.shape, cst), pl.BlockSpec(hb2.shape, cst),
            pl.BlockSpec(gwa.shape, cst), pl.BlockSpec(gwb.shape, cst),
            pl.BlockSpec(gwc.shape, cst), pl.BlockSpec(gb1.shape, cst),
            pl.BlockSpec(gw2.shape, cst), pl.BlockSpec(gb2.shape, cst),
            pl.BlockSpec(gw3.shape, cst), pl.BlockSpec(gb3.shape, cst),
        ],
        out_specs=[nd32, pl.BlockSpec((64, 32), cst)],
        out_shape=[jax.ShapeDtypeStruct((n, 32), F32),
                   jax.ShapeDtypeStruct((64, 32), F32)],
        scratch_shapes=[pltpu.VMEM((64, 32), F32), pltpu.VMEM((64, 32), F32),
                        pltpu.VMEM((64, 8), F32), pltpu.VMEM((64, 8), F32)],
    )(seg4, seg4, cnt8, cnt8, xtab, ubtab, x_res, u_src, batch_r3,
      wn1, bn1, wn2, bn2, wn3, bn3, hw1, hb1, hw2, hb2,
      gwa, gwb, gwc, gb1, gw2, gb2, gw3, gb3)


def _lin(layer):
    w, b = layer
    return w, b.reshape(1, -1)


def _diag2(w):
    dk, dn = w.shape
    z = jnp.zeros((dk, dn), F32)
    return jnp.concatenate([
        jnp.concatenate([w, z], axis=1),
        jnp.concatenate([z, w], axis=1),
    ], axis=0)


def _dup2(b):
    return jnp.concatenate([b, b], axis=1)


def _run(x, edge_index, edge_attr, u, batch, params):
    n = x.shape[0]
    e = edge_index.shape[1]
    row = edge_index[0].astype(I32)
    col = edge_index[1].astype(I32)
    batch32 = batch.astype(I32)
    batch_col = batch32[:, None]
    batch_r3 = batch32.reshape(n // TN, 1, TN)
    zeros32 = jnp.zeros((n, 32), F32)
    zeros16 = jnp.zeros((n, 16), F32)
    ones16 = jnp.ones((CH, 16), F32)

    cnt8 = _sc_counts(row, n, zeros16, ones16)

    nfw1, nfb1 = _lin(params['node_dense_first'][0])
    nfw2, nfb2 = _lin(params['node_dense_first'][1])
    gfw1, gfb1 = _lin(params['global_dense_first'][0])
    gfw2, gfb2 = _lin(params['global_dense_first'][1])
    efw1, efb1 = _lin(params['edge_dense_first'][0])
    efw2, efb2 = _lin(params['edge_dense_first'][1])

    x_out = None
    edge_out = None
    u_out = None
    for i in range(3):
        mp = params['megnet'][i]
        if i == 0:
            x_in, u_src, e_src = x, u, edge_attr
            nw = (nfw1, nfb1, nfw2, nfb2)
            gw = (gfw1, gfb1, gfw2, gfb2)
            ew = (efw1, efb1, efw2, efb2)
        else:
            x_in, u_src, e_src = x_out, u_out, edge_out
            nd1, nd2 = params['node_dense'][i - 1]
            gd1, gd2 = params['global_dense'][i - 1]
            ed1, ed2 = params['edge_dense'][i - 1]
            nw = _lin(nd1) + _lin(nd2)
            gw = _lin(gd1) + _lin(gd2)
            ew = _lin(ed1) + _lin(ed2)

        ew1m, eb1m = _lin(mp['edge_mlp'][0])
        ew2m, eb2m = _lin(mp['edge_mlp'][1])
        ew3m, eb3m = _lin(mp['edge_mlp'][2])
        wxr, wxc, weh, wub = (ew1m[0:32], ew1m[32:64], ew1m[64:96],
                              ew1m[96:128])

        xtab, ubtab, atab, ctab = _tc_prep(
            x_in, u_src, batch_col, *nw, *gw, wxr, wxc, wub, eb1m)
        ga, gc = _sc_gather(atab, ctab, row, col)

        ew1, eb1, ew2, eb2 = ew
        es2 = e_src if i > 0 else e_src.reshape(e // 2, 32)
        eh2, edge_out = _tc_edge(
            es2, ga.reshape(e // 2, 128), gc.reshape(e // 2, 128),
            _diag2(ew1), _dup2(eb1), _diag2(ew2), _dup2(eb2),
            _diag2(weh), _diag2(ew2m), _dup2(eb2m), _diag2(ew3m),
            _dup2(eb3m), block0=(i == 0))

        seg4 = _sc_scatter_add(eh2.reshape(e, 32), row, n, zeros32)

        nw1m, nb1m = _lin(mp['node_mlp'][0])
        nw2m, nb2m = _lin(mp['node_mlp'][1])
        nw3m, nb3m = _lin(mp['node_mlp'][2])
        gw1m, gb1m = _lin(mp['global_mlp'][0])
        gw2m, gb2m = _lin(mp['global_mlp'][1])
        gw3m, gb3m = _lin(mp['global_mlp'][2])
        gwa, gwb, gwc = gw1m[0:32], gw1m[32:64], gw1m[64:96]

        x_res = xtab if i == 0 else x_in
        x_out, u_out = _tc_node_global(
            seg4, cnt8, xtab, ubtab, x_res, u_src, batch_r3,
            nw1m, nb1m, nw2m, nb2m, nw3m, nb3m,
            *gw, gwa, gwb, gwc, gb1m, gw2m, gb2m, gw3m, gb3m,
            block0=(i == 0))

    return (x_out, edge_out.reshape(e, 32), u_out)


_run_jit = jax.jit(_run)


def kernel(x, edge_index, edge_attr, u, batch, params):
    return _run_jit(x, edge_index, edge_attr, u, batch, params)


# speedup vs baseline: 7.5052x; 1.0073x over previous
import functools

import jax
import jax.numpy as jnp
from jax import lax
from jax.experimental import pallas as pl
from jax.experimental.pallas import tpu as pltpu
from jax.experimental.pallas import tpu_sc as plsc

F32 = jnp.float32
I32 = jnp.int32

NC = 2
NS = 16
NW = NC * NS

TN = 2000
TE = 16000
CH = 1000

_SC_PARAMS = pltpu.CompilerParams(use_tc_tiling_on_sc=False)
_SC_PARAMS_BIG = pltpu.CompilerParams(use_tc_tiling_on_sc=False,
                                      internal_scratch_in_bytes=0)


def _relu(v):
    return jnp.maximum(v, 0.0)


def _dot(a, b):
    return jnp.dot(a, b, preferred_element_type=F32)


def _mlp2(v, w1, b1, w2, b2):
    return _dot(_relu(_dot(v, w1) + b1), w2) + b2


def _sc_gather(atab, ctab, row, col):
    e = row.shape[0]
    ew = e // NW
    chg = 200
    nch = ew // chg
    npairs = (nch - 1) // 2
    mesh = plsc.VectorSubcoreMesh(core_axis_name="c", subcore_axis_name="s")
    out_t = jax.ShapeDtypeStruct((e, 64), F32)

    @functools.partial(
        pl.kernel, mesh=mesh,
        compiler_params=_SC_PARAMS,
        out_type=(out_t, out_t),
        scratch_types=[
            pltpu.VMEM((chg,), I32), pltpu.VMEM((chg,), I32),
            pltpu.VMEM((chg,), I32), pltpu.VMEM((chg,), I32),
            pltpu.VMEM((chg, 64), F32), pltpu.VMEM((chg, 64), F32),
            pltpu.VMEM((chg, 64), F32), pltpu.VMEM((chg, 64), F32),
            pltpu.SemaphoreType.DMA, pltpu.SemaphoreType.DMA,
        ],
    )
    def k(at_hbm, ct_hbm, row_hbm, col_hbm, ga_hbm, gc_hbm,
          ir0, ir1, ic0, ic1, ba0, ba1, bc0, bc1, sem0, sem1):
        wid = lax.axis_index("s") * NC + lax.axis_index("c")
        base = wid * ew
        irs = (ir0, ir1)
        ics = (ic0, ic1)
        bas = (ba0, ba1)
        bcs = (bc0, bc1)
        sems = (sem0, sem1)

        def load_and_fire(off, s):
            pltpu.sync_copy(row_hbm.at[pl.ds(off, chg)], irs[s])
            pltpu.sync_copy(col_hbm.at[pl.ds(off, chg)], ics[s])
            pltpu.async_copy(at_hbm.at[irs[s]], bas[s], sems[s])
            pltpu.async_copy(ct_hbm.at[ics[s]], bcs[s], sems[s])

        def drain_and_store(off, s):
            pltpu.make_async_copy(at_hbm.at[irs[s]], bas[s], sems[s]).wait()
            pltpu.make_async_copy(ct_hbm.at[ics[s]], bcs[s], sems[s]).wait()
            pltpu.sync_copy(bas[s], ga_hbm.at[pl.ds(off, chg)])
            pltpu.sync_copy(bcs[s], gc_hbm.at[pl.ds(off, chg)])

        load_and_fire(base, 0)

        @pl.loop(0, npairs)
        def _(j):
            c0 = base + (2 * j) * chg
            load_and_fire(c0 + chg, 1)
            drain_and_store(c0, 0)
            load_and_fire(c0 + 2 * chg, 0)
            drain_and_store(c0 + chg, 1)

        drain_and_store(base + (nch - 1) * chg, 0)

    return k(atab, ctab, row, col)


def _sc_scatter_add(vals, row, n, zeros32):
    e = vals.shape[0]
    ew = e // NW
    chs = 200
    nch = ew // chs
    npairs = (nch - 1) // 2
    nps = n // NS
    mesh = plsc.VectorSubcoreMesh(core_axis_name="c", subcore_axis_name="s")

    @functools.partial(
        pl.kernel, mesh=mesh,
        compiler_params=_SC_PARAMS_BIG,
        out_type=jax.ShapeDtypeStruct((NC, n, 32), F32),
        scratch_types=[
            pltpu.VMEM((chs,), I32), pltpu.VMEM((chs,), I32),
            pltpu.VMEM((chs, 32), F32), pltpu.VMEM((chs, 32), F32),
            pltpu.VMEM_SHARED((n, 32), F32),
            pltpu.SemaphoreType.DMA, pltpu.SemaphoreType.DMA,
        ],
    )
    def k(vals_hbm, row_hbm, z_hbm, out_hbm, ix0, ix1, va0, va1, shared,
          sem0, sem1):
        cid = lax.axis_index("c")
        sid = lax.axis_index("s")
        pltpu.sync_copy(z_hbm.at[pl.ds(sid * nps, nps)],
                        shared.at[pl.ds(sid * nps, nps)])
        plsc.subcore_barrier()
        base = cid * (ew * NS) + sid * ew
        ixs = (ix0, ix1)
        vas = (va0, va1)
        sems = (sem0, sem1)

        def fire(off, s):
            pltpu.async_copy(row_hbm.at[pl.ds(off, chs)], ixs[s], sems[s])
            pltpu.async_copy(vals_hbm.at[pl.ds(off, chs)], vas[s], sems[s])

        def scat(off, s):
            pltpu.make_async_copy(row_hbm.at[pl.ds(off, chs)], ixs[s],
                                  sems[s]).wait()
            pltpu.make_async_copy(vals_hbm.at[pl.ds(off, chs)], vas[s],
                                  sems[s]).wait()
            pltpu.sync_copy(vas[s], shared.at[ixs[s]], add=True)

        fire(base, 0)

        @pl.loop(0, npairs)
        def _(j):
            c0 = base + (2 * j) * chs
            fire(c0 + chs, 1)
            scat(c0, 0)
            fire(c0 + 2 * chs, 0)
            scat(c0 + chs, 1)

        scat(base + (nch - 1) * chs, 0)
        plsc.subcore_barrier()
        pltpu.sync_copy(shared.at[pl.ds(sid * nps, nps)],
                        out_hbm.at[cid, pl.ds(sid * nps, nps)])

    return k(vals, row, zeros32)


def _sc_counts(row, n, zeros16, ones16):
    e = row.shape[0]
    ew = e // NW
    nch = ew // CH
    nps = n // NS
    mesh = plsc.VectorSubcoreMesh(core_axis_name="c", subcore_axis_name="s")

    @functools.partial(
        pl.kernel, mesh=mesh,
        compiler_params=_SC_PARAMS,
        out_type=jax.ShapeDtypeStruct((NC, n, 16), F32),
        scratch_types=[
            pltpu.VMEM((CH,), I32), pltpu.VMEM((CH, 16), F32),
            pltpu.VMEM_SHARED((n, 16), F32),
        ],
    )
    def k(row_hbm, z_hbm, ones_hbm, out_hbm, idx_v, ones_v, shared):
        cid = lax.axis_index("c")
        sid = lax.axis_index("s")
        pltpu.sync_copy(z_hbm.at[pl.ds(sid * nps, nps)],
                        shared.at[pl.ds(sid * nps, nps)])
        pltpu.sync_copy(ones_hbm, ones_v)
        plsc.subcore_barrier()
        base = cid * (ew * NS) + sid * ew

        @pl.loop(0, nch)
        def _(kk):
            off = base + kk * CH
            pltpu.sync_copy(row_hbm.at[pl.ds(off, CH)], idx_v)
            pltpu.sync_copy(ones_v, shared.at[idx_v], add=True)

        plsc.subcore_barrier()
        pltpu.sync_copy(shared.at[pl.ds(sid * nps, nps)],
                        out_hbm.at[cid, pl.ds(sid * nps, nps)])

    return k(row, zeros16, ones16)


def _tc_prep(x_in, u_src, batch_col, nw1, nb1, nw2, nb2, gw1, gb1, gw2, gb2,
             wxr, wxc, wub, be1):
    n, din = x_in.shape
    grid = n // TN

    def body(x_ref, u_ref, b_ref, nw1r, nb1r, nw2r, nb2r,
             gw1r, gb1r, gw2r, gb2r, wxrr, wxcr, wubr, be1r,
             xt_ref, ub_ref, at_ref, ct_ref):
        xh = _mlp2(x_ref[...], nw1r[...], nb1r[...], nw2r[...], nb2r[...])
        u_head = _mlp2(u_ref[...], gw1r[...], gb1r[...], gw2r[...], gb2r[...])
        oh = (b_ref[...] == lax.broadcasted_iota(I32, (1, 64), 1)).astype(F32)
        ub = _dot(oh, u_head)
        xt_ref[...] = xh
        ub_ref[...] = ub
        at_ref[...] = _dot(xh, wxrr[...]) + _dot(ub, wubr[...]) + be1r[...]
        ct_ref[...] = _dot(xh, wxcr[...])

    cst = lambda *_: (0, 0)
    out32 = jax.ShapeDtypeStruct((n, 32), F32)
    outp = jax.ShapeDtypeStruct((n, 64), F32)
    return pl.pallas_call(
        body,
        grid=(grid,),
        in_specs=[
            pl.BlockSpec((TN, din), lambda i: (i, 0)),
            pl.BlockSpec((64, 32), cst),
            pl.BlockSpec((TN, 1), lambda i: (i, 0)),
            pl.BlockSpec(nw1.shape, cst), pl.BlockSpec(nb1.shape, cst),
            pl.BlockSpec(nw2.shape, cst), pl.BlockSpec(nb2.shape, cst),
            pl.BlockSpec(gw1.shape, cst), pl.BlockSpec(gb1.shape, cst),
            pl.BlockSpec(gw2.shape, cst), pl.BlockSpec(gb2.shape, cst),
            pl.BlockSpec(wxr.shape, cst), pl.BlockSpec(wxc.shape, cst),
            pl.BlockSpec(wub.shape, cst), pl.BlockSpec(be1.shape, cst),
        ],
        out_specs=[pl.BlockSpec((TN, 32), lambda i: (i, 0)),
                   pl.BlockSpec((TN, 32), lambda i: (i, 0)),
                   pl.BlockSpec((TN, 64), lambda i: (i, 0)),
                   pl.BlockSpec((TN, 64), lambda i: (i, 0))],
        out_shape=[out32, out32, outp, outp],
    )(x_in, u_src, batch_col, nw1, nb1, nw2, nb2, gw1, gb1, gw2, gb2,
      wxr, wxc, wub, be1)


def _tc_edge(es2, ga2, gc2, ew1d, eb1d, ew2d, eb2d,
             wehd, we2dd, be2dd, we3dd, be3dd, block0):
    eh_, din2 = es2.shape
    grid = 2 * eh_ // TE
    th = TE // 2

    def body(es_ref, ga_ref, gc_ref, ew1r, eb1r, ew2r, eb2r,
             wehr, we2dr, be2dr, we3dr, be3dr, eh_ref, eo_ref):
        es = es_ref[...]
        e0 = _mlp2(es, ew1r[...], eb1r[...], ew2r[...], eb2r[...])
        z = _relu(ga_ref[...] + gc_ref[...] + _dot(e0, wehr[...]))
        z = _relu(_dot(z, we2dr[...]) + be2dr[...])
        ehp = _dot(z, we3dr[...]) + be3dr[...]
        eh_ref[...] = ehp
        eo_ref[...] = (e0 if block0 else es) + ehp

    cst = lambda *_: (0, 0)
    pk = pl.BlockSpec((th, 128), lambda i: (i, 0))
    p64 = pl.BlockSpec((th, 64), lambda i: (i, 0))
    return pl.pallas_call(
        body,
        grid=(grid,),
        in_specs=[
            pl.BlockSpec((th, din2), lambda i: (i, 0)), pk, pk,
            pl.BlockSpec(ew1d.shape, cst), pl.BlockSpec(eb1d.shape, cst),
            pl.BlockSpec(ew2d.shape, cst), pl.BlockSpec(eb2d.shape, cst),
            pl.BlockSpec(wehd.shape, cst),
            pl.BlockSpec(we2dd.shape, cst), pl.BlockSpec(be2dd.shape, cst),
            pl.BlockSpec(we3dd.shape, cst), pl.BlockSpec(be3dd.shape, cst),
        ],
        out_specs=[p64, p64],
        out_shape=[jax.ShapeDtypeStruct((eh_, 64), F32),
                   jax.ShapeDtypeStruct((eh_, 64), F32)],
    )(es2, ga2, gc2, ew1d, eb1d, ew2d, eb2d, wehd, we2dd, be2dd,
      we3dd, be3dd)


def _tc_node_global(seg4, cnt8, xtab, ubtab, x_res, u_src, batch_r3,
                    wn1, bn1, wn2, bn2, wn3, bn3,
                    hw1, hb1, hw2, hb2,
                    gwa, gwb, gwc, gb1, gw2, gb2, gw3, gb3,
                    block0):
    n = xtab.shape[0]
    grid = n // TN

    def body(s0_ref, s1_ref, c0_ref, c1_ref, xt_ref, ub_ref, xr_ref, u_ref,
             br_ref, wn1r, bn1r, wn2r, bn2r, wn3r, bn3r,
             hw1r, hb1r, hw2r, hb2r,
             gwar, gwbr, gwcr, gb1r, gw2r, gb2r, gw3r, gb3r,
             xo_ref, uo_ref, gn_acc, ge_acc, cn_acc, ce_acc):
        i = pl.program_id(0)

        @pl.when(i == 0)
        def _():
            gn_acc[...] = jnp.zeros_like(gn_acc)
            ge_acc[...] = jnp.zeros_like(ge_acc)
            cn_acc[...] = jnp.zeros_like(cn_acc)
            ce_acc[...] = jnp.zeros_like(ce_acc)

        deg = c0_ref[0] + c1_ref[0]
        s = s0_ref[0] + s1_ref[0]
        agg = s * (1.0 / jnp.maximum(deg[:, 0:1], 1.0))
        xh = xt_ref[...]
        ub = ub_ref[...]
        nin = jnp.concatenate([agg, xh, ub], axis=1)
        h = _relu(_dot(nin, wn1r[...]) + bn1r[...])
        h = _relu(_dot(h, wn2r[...]) + bn2r[...])
        xh_new = _dot(h, wn3r[...]) + bn3r[...]
        xo_ref[...] = xr_ref[...] + xh_new

        oht = (lax.broadcasted_iota(I32, (64, 1), 0)
               == br_ref[0]).astype(F32)
        gn_acc[...] += _dot(oht, xh_new)
        ge_acc[...] += _dot(oht, s)
        cn_acc[...] += _dot(oht, jnp.ones((TN, 8), F32))
        ce_acc[...] += _dot(oht, deg[:, 0:8])

        @pl.when(i == grid - 1)
        def _():
            u_head = _mlp2(u_ref[...], hw1r[...], hb1r[...],
                           hw2r[...], hb2r[...])
            node_mean = gn_acc[...] / jnp.maximum(cn_acc[...][:, 0:1], 1.0)
            edge_mean = ge_acc[...] / jnp.maximum(ce_acc[...][:, 0:1], 1.0)
            g = _relu(_dot(u_head, gwar[...]) + _dot(node_mean, gwbr[...])
                      + _dot(edge_mean, gwcr[...]) + gb1r[...])
            g = _relu(_dot(g, gw2r[...]) + gb2r[...])
            uh_new = _dot(g, gw3r[...]) + gb3r[...]
            u_base = u_head if block0 else u_ref[...]
            uo_ref[...] = u_base + uh_new

    cst = lambda *_: (0, 0)
    nd32 = pl.BlockSpec((TN, 32), lambda i: (i, 0))
    sspec0 = pl.BlockSpec((1, TN, 32), lambda i: (0, i, 0))
    sspec1 = pl.BlockSpec((1, TN, 32), lambda i: (1, i, 0))
    cspec0 = pl.BlockSpec((1, TN, 16), lambda i: (0, i, 0))
    cspec1 = pl.BlockSpec((1, TN, 16), lambda i: (1, i, 0))
    return pl.pallas_call(
        body,
        grid=(grid,),
        in_specs=[
            sspec0, sspec1, cspec0, cspec1, nd32, nd32, nd32,
            pl.BlockSpec((64, 32), cst),
            pl.BlockSpec((1, 1, TN), lambda i: (i, 0, 0)),
            pl.BlockSpec(wn1.shape, cst), pl.BlockSpec(bn1.shape, cst),
            pl.BlockSpec(wn2.shape, cst), pl.BlockSpec(bn2.shape, cst),
            pl.BlockSpec(wn3.shape, cst), pl.BlockSpec(bn3.shape, cst),
            pl.BlockSpec(hw1.shape, cst), pl.BlockSpec(hb1.shape, cst),
            pl.BlockSpec(hw2.shape, cst), pl.BlockSpec(hb2.shape, cst),
            pl.BlockSpec(gwa.shape, cst), pl.BlockSpec(gwb.shape, cst),
            pl.BlockSpec(gwc.shape, cst), pl.BlockSpec(gb1.shape, cst),
            pl.BlockSpec(gw2.shape, cst), pl.BlockSpec(gb2.shape, cst),
            pl.BlockSpec(gw3.shape, cst), pl.BlockSpec(gb3.shape, cst),
        ],
        out_specs=[nd32, pl.BlockSpec((64, 32), cst)],
        out_shape=[jax.ShapeDtypeStruct((n, 32), F32),
                   jax.ShapeDtypeStruct((64, 32), F32)],
        scratch_shapes=[pltpu.VMEM((64, 32), F32), pltpu.VMEM((64, 32), F32),
                        pltpu.VMEM((64, 8), F32), pltpu.VMEM((64, 8), F32)],
    )(seg4, seg4, cnt8, cnt8, xtab, ubtab, x_res, u_src, batch_r3,
      wn1, bn1, wn2, bn2, wn3, bn3, hw1, hb1, hw2, hb2,
      gwa, gwb, gwc, gb1, gw2, gb2, gw3, gb3)


def _lin(layer):
    w, b = layer
    return w, b.reshape(1, -1)


def _diag2(w):
    dk, dn = w.shape
    z = jnp.zeros((dk, dn), F32)
    return jnp.concatenate([
        jnp.concatenate([w, z], axis=1),
        jnp.concatenate([z, w], axis=1),
    ], axis=0)


def _dup2(b):
    return jnp.concatenate([b, b], axis=1)


def _run(x, edge_index, edge_attr, u, batch, params):
    n = x.shape[0]
    e = edge_index.shape[1]
    row = edge_index[0].astype(I32)
    col = edge_index[1].astype(I32)
    batch32 = batch.astype(I32)
    batch_col = batch32[:, None]
    batch_r3 = batch32.reshape(n // TN, 1, TN)
    zeros32 = jnp.zeros((n, 32), F32)
    zeros16 = jnp.zeros((n, 16), F32)
    ones16 = jnp.ones((CH, 16), F32)

    cnt8 = _sc_counts(row, n, zeros16, ones16)

    nfw1, nfb1 = _lin(params['node_dense_first'][0])
    nfw2, nfb2 = _lin(params['node_dense_first'][1])
    gfw1, gfb1 = _lin(params['global_dense_first'][0])
    gfw2, gfb2 = _lin(params['global_dense_first'][1])
    efw1, efb1 = _lin(params['edge_dense_first'][0])
    efw2, efb2 = _lin(params['edge_dense_first'][1])

    x_out = None
    edge_out = None
    u_out = None
    for i in range(3):
        mp = params['megnet'][i]
        if i == 0:
            x_in, u_src, e_src = x, u, edge_attr
            nw = (nfw1, nfb1, nfw2, nfb2)
            gw = (gfw1, gfb1, gfw2, gfb2)
            ew = (efw1, efb1, efw2, efb2)
        else:
            x_in, u_src, e_src = x_out, u_out, edge_out
            nd1, nd2 = params['node_dense'][i - 1]
            gd1, gd2 = params['global_dense'][i - 1]
            ed1, ed2 = params['edge_dense'][i - 1]
            nw = _lin(nd1) + _lin(nd2)
            gw = _lin(gd1) + _lin(gd2)
            ew = _lin(ed1) + _lin(ed2)

        ew1m, eb1m = _lin(mp['edge_mlp'][0])
        ew2m, eb2m = _lin(mp['edge_mlp'][1])
        ew3m, eb3m = _lin(mp['edge_mlp'][2])
        wxr, wxc, weh, wub = (ew1m[0:32], ew1m[32:64], ew1m[64:96],
                              ew1m[96:128])

        xtab, ubtab, atab, ctab = _tc_prep(
            x_in, u_src, batch_col, *nw, *gw, wxr, wxc, wub, eb1m)
        ga, gc = _sc_gather(atab, ctab, row, col)

        ew1, eb1, ew2, eb2 = ew
        es2 = e_src if i > 0 else e_src.reshape(e // 2, 32)
        eh2, edge_out = _tc_edge(
            es2, ga.reshape(e // 2, 128), gc.reshape(e // 2, 128),
            _diag2(ew1), _dup2(eb1), _diag2(ew2), _dup2(eb2),
            _diag2(weh), _diag2(ew2m), _dup2(eb2m), _diag2(ew3m),
            _dup2(eb3m), block0=(i == 0))

        seg4 = _sc_scatter_add(eh2.reshape(e, 32), row, n, zeros32)

        nw1m, nb1m = _lin(mp['node_mlp'][0])
        nw2m, nb2m = _lin(mp['node_mlp'][1])
        nw3m, nb3m = _lin(mp['node_mlp'][2])
        gw1m, gb1m = _lin(mp['global_mlp'][0])
        gw2m, gb2m = _lin(mp['global_mlp'][1])
        gw3m, gb3m = _lin(mp['global_mlp'][2])
        gwa, gwb, gwc = gw1m[0:32], gw1m[32:64], gw1m[64:96]

        x_res = xtab if i == 0 else x_in
        x_out, u_out = _tc_node_global(
            seg4, cnt8, xtab, ubtab, x_res, u_src, batch_r3,
            nw1m, nb1m, nw2m, nb2m, nw3m, nb3m,
            *gw, gwa, gwb, gwc, gb1m, gw2m, gb2m, gw3m, gb3m,
            block0=(i == 0))

    return (x_out, edge_out.reshape(e, 32), u_out)


_run_jit = jax.jit(_run)


def kernel(x, edge_index, edge_attr, u, batch, params):
    return _run_jit(x, edge_index, edge_attr, u, batch, params)
